# baseline (speedup 1.0000x reference)
import functools

import jax
import jax.numpy as jnp
from jax import lax
from jax.experimental import pallas as pl
from jax.experimental.pallas import tpu as pltpu
from jax.experimental.pallas import tpu_sc as plsc

_NC = 2
_NS = 16
_LANES = 16


def _lane_bcast(v, l):
    idx = jnp.full((_LANES, 1), l, jnp.int32)
    dn = lax.GatherDimensionNumbers(
        offset_dims=(), collapsed_slice_dims=(0,), start_index_map=(0,))
    return lax.gather(v, idx, dn, slice_sizes=(1,),
                      mode=lax.GatherScatterMode.PROMISE_IN_BOUNDS)


def _gate_body(ef_ref, w_ref, out_ref):
    ef = ef_ref[...].reshape(ef_ref.shape[0] // 128, 128)
    z = jnp.dot(ef, w_ref[...], preferred_element_type=jnp.float32)
    out_ref[...] = jax.nn.sigmoid(z).reshape(out_ref.shape)


def _compute_gate(edge_feat, w_gate):
    E, DE = edge_feat.shape
    P = 128
    U = E // 8
    G = 20
    BR = U // G
    assert BR * G == U and BR % 8 == 0
    ef_flat = edge_feat.reshape(U * P)
    wrep = jnp.tile(jnp.kron(jnp.eye(8, dtype=jnp.float32),
                             w_gate.reshape(DE, 1)), (1, 16))
    return pl.pallas_call(
        _gate_body,
        grid=(G,),
        in_specs=[
            pl.BlockSpec((BR * P,), lambda i: (i,)),
            pl.BlockSpec((P, P), lambda i: (0, 0)),
        ],
        out_specs=pl.BlockSpec((BR * P,), lambda i: (i,)),
        out_shape=jax.ShapeDtypeStruct((U * P,), jnp.float32),
    )(ef_flat, wrep)


def _make_sc_agg(N, D, E, e_base):
    NW = _NC * _NS
    EW = E // NW
    C = 80
    CH = EW // C
    RT = 624
    TAIL = N - _NS * RT
    GRP = C // _LANES
    JD = D // _LANES
    ZC = RT // C
    ZT = RT - ZC * C
    assert EW * NW == E and CH * C == EW and CH >= 8
    assert TAIL == 16 and ZT == 64

    mesh = plsc.VectorSubcoreMesh(core_axis_name="c", subcore_axis_name="s")

    @functools.partial(
        pl.kernel,
        out_type=jax.ShapeDtypeStruct((_NC, N, D), jnp.float32),
        mesh=mesh,
        scratch_types=[
            [pltpu.VMEM((16 * C,), jnp.float32) for _ in range(4)],
            [pltpu.VMEM((C,), jnp.int32) for _ in range(4)],
            [pltpu.VMEM((C,), jnp.int32) for _ in range(4)],
            [pltpu.VMEM((C, D), jnp.float32) for _ in range(4)],
            pltpu.VMEM_SHARED((N, D), jnp.float32),
            [pltpu.SemaphoreType.DMA for _ in range(4)],
            [pltpu.SemaphoreType.DMA for _ in range(4)],
            [pltpu.SemaphoreType.DMA for _ in range(4)],
            [pltpu.SemaphoreType.DMA for _ in range(4)],
            [pltpu.SemaphoreType.DMA for _ in range(4)],
        ],
    )
    def sc_agg(src_hbm, dst_hbm, gate_hbm, x_hbm, out_hbm,
               gatev, srcv, dstv, rows, agg_sh,
               sem_gt, sem_src, sem_dst, sem_g, sem_s):
        c = lax.axis_index("c")
        s = lax.axis_index("s")
        wid = s * _NC + c
        base_w = wid * EW
        base_g = e_base + base_w

        zeros = jnp.zeros((_LANES,), jnp.float32)

        def zrow(r, carry):
            for j in range(JD):
                rows[0][r, pl.ds(j * _LANES, _LANES)] = zeros
            return carry

        lax.fori_loop(0, C, zrow, 0)
        for i in range(ZC):
            pltpu.async_copy(rows[0], agg_sh.at[pl.ds(s * RT + i * C, C)], sem_s[i % 4])
        pltpu.async_copy(rows[0].at[pl.ds(0, ZT)],
                         agg_sh.at[pl.ds(s * RT + ZC * C, ZT)], sem_s[3])

        @pl.when(s == 0)
        def _zero_tail():
            pltpu.sync_copy(rows[0].at[pl.ds(0, TAIL)], agg_sh.at[pl.ds(_NS * RT, TAIL)])

        for i in range(ZC):
            pltpu.make_async_copy(rows[0], agg_sh.at[pl.ds(s * RT + i * C, C)],
                                  sem_s[i % 4]).wait()
        pltpu.make_async_copy(rows[0].at[pl.ds(0, ZT)],
                              agg_sh.at[pl.ds(s * RT + ZC * C, ZT)], sem_s[3]).wait()
        plsc.subcore_barrier()

        def start_src(k, p):
            pltpu.async_copy(src_hbm.at[pl.ds(base_g + k * C, C)], srcv[p], sem_src[p])

        def wait_src(k, p):
            pltpu.make_async_copy(src_hbm.at[pl.ds(base_g + k * C, C)],
                                  srcv[p], sem_src[p]).wait()

        def start_dst(k, p):
            pltpu.async_copy(dst_hbm.at[pl.ds(base_g + k * C, C)], dstv[p], sem_dst[p])

        def wait_dst(k, p):
            pltpu.make_async_copy(dst_hbm.at[pl.ds(base_g + k * C, C)],
                                  dstv[p], sem_dst[p]).wait()

        def start_gate(k, p):
            pltpu.async_copy(gate_hbm.at[pl.ds(16 * (base_w + k * C), 16 * C)],
                             gatev[p], sem_gt[p])

        def wait_gate(k, p):
            pltpu.make_async_copy(gate_hbm.at[pl.ds(16 * (base_w + k * C), 16 * C)],
                                  gatev[p], sem_gt[p]).wait()

        def start_gather(p):
            pltpu.async_copy(x_hbm.at[srcv[p]], rows[p], sem_g[p])

        def wait_gather(p):
            pltpu.make_async_copy(x_hbm.at[srcv[p]], rows[p], sem_g[p]).wait()

        def start_scatter(p):
            pltpu.async_copy(rows[p], agg_sh.at[dstv[p]], sem_s[p], add=True)

        def wait_scatter(p):
            pltpu.make_async_copy(rows[p], agg_sh.at[dstv[p]], sem_s[p]).wait()

        def scale(p):
            def grp(q, gcarry):
                ga = gatev[p][pl.ds(q * 256, _LANES)]
                gb = gatev[p][pl.ds(q * 256 + 128, _LANES)]
                for l in range(_LANES):
                    g16 = _lane_bcast(ga if l < 8 else gb, l % 8)
                    e = q * _LANES + l
                    for j in range(JD):
                        sl = pl.ds(j * _LANES, _LANES)
                        rows[p][e, sl] = rows[p][e, sl] * g16
                return gcarry

            lax.fori_loop(0, GRP, grp, 0)

        for j in range(4):
            start_src(j, j)
            start_dst(j, j)
            start_gate(j, j)
        for j in range(4):
            wait_src(j, j)
            start_gather(j)

        def head(k, p):
            wait_gather(p)
            wait_gate(k, p)
            scale(p)
            wait_dst(k, p)
            start_scatter(p)

        head(0, 0)
        start_src(4, 0)
        head(1, 1)
        head(2, 2)

        def full_step(k, p, pn, pn2, with_next2):
            wait_scatter(pn)
            start_dst(k + 1, pn)
            start_gate(k + 1, pn)
            if with_next2:
                start_src(k + 2, pn2)
            wait_src(k + 1, pn)
            start_gather(pn)
            wait_gather(p)
            wait_gate(k, p)
            scale(p)
            wait_dst(k, p)
            start_scatter(p)

        T = (CH - 5) // 4

        def quad(t, carry):
            k0 = 4 * t + 3
            full_step(k0, 3, 0, 1, True)
            full_step(k0 + 1, 0, 1, 2, True)
            full_step(k0 + 2, 1, 2, 3, True)
            full_step(k0 + 3, 2, 3, 0, True)
            return carry

        lax.fori_loop(0, T, quad, 0)
        for k in range(3 + 4 * T, CH):
            p, pn, pn2 = k % 4, (k + 1) % 4, (k + 2) % 4
            if k < CH - 1:
                full_step(k, p, pn, pn2, k + 2 < CH)
            else:
                wait_scatter(pn)
                wait_gather(p)
                wait_gate(k, p)
                scale(p)
                wait_dst(k, p)
                start_scatter(p)
        for k in range(CH - 3, CH):
            wait_scatter(k % 4)
        plsc.subcore_barrier()

        pltpu.sync_copy(agg_sh.at[pl.ds(s * RT, RT)], out_hbm.at[c, pl.ds(s * RT, RT)])

        @pl.when(s == 0)
        def _write_tail():
            pltpu.sync_copy(agg_sh.at[pl.ds(_NS * RT, TAIL)],
                            out_hbm.at[c, pl.ds(_NS * RT, TAIL)])

    return sc_agg


def _out_body(a0_ref, a1_ref, a2_ref, a3_ref, x_ref, wn_ref, ws_ref, b_ref, o_ref):
    agg = (a0_ref[...] + a1_ref[...]) + (a2_ref[...] + a3_ref[...])
    acc = jnp.dot(agg, wn_ref[...], preferred_element_type=jnp.float32)
    acc = acc + jnp.dot(x_ref[...], ws_ref[...], preferred_element_type=jnp.float32)
    acc = acc + b_ref[...]
    o_ref[...] = jnp.maximum(acc, 0.0)


def _compute_out(aggs, x, W_neigh, W_self, b):
    N, D = x.shape
    R = 1000
    G = N // R
    b_row = b.reshape(1, D)
    blk = pl.BlockSpec((R, D), lambda i: (i, 0))
    return pl.pallas_call(
        _out_body,
        grid=(G,),
        in_specs=[
            blk, blk, blk, blk,
            pl.BlockSpec((R, D), lambda i: (i, 0)),
            pl.BlockSpec((D, D), lambda i: (0, 0)),
            pl.BlockSpec((D, D), lambda i: (0, 0)),
            pl.BlockSpec((1, D), lambda i: (0, 0)),
        ],
        out_specs=pl.BlockSpec((R, D), lambda i: (i, 0)),
        out_shape=jax.ShapeDtypeStruct((N, D), jnp.float32),
    )(*aggs, x, W_neigh, W_self, b_row)


@jax.jit
def kernel(g, node_feat, edge_feat, embed_weight, W_self, W_neigh, w_gate, b):
    N, D = embed_weight.shape
    E = edge_feat.shape[0]
    src = g[0]
    dst = g[1]
    E0 = 128000
    parts = []
    for lo, hi in ((0, E0), (E0, E)):
        gate = _compute_gate(edge_feat[lo:hi], w_gate)
        aggs = _make_sc_agg(N, D, hi - lo, lo)(src, dst, gate, embed_weight)
        parts.append(aggs[0])
        parts.append(aggs[1])
    return _compute_out(parts, embed_weight, W_neigh, W_self, b)


# speedup vs baseline: 1.0265x; 1.0265x over previous
import functools

import jax
import jax.numpy as jnp
from jax import lax
from jax.experimental import pallas as pl
from jax.experimental.pallas import tpu as pltpu
from jax.experimental.pallas import tpu_sc as plsc

_NC = 2
_NS = 16
_LANES = 16


def _lane_bcast(v, l):
    idx = jnp.full((_LANES, 1), l, jnp.int32)
    dn = lax.GatherDimensionNumbers(
        offset_dims=(), collapsed_slice_dims=(0,), start_index_map=(0,))
    return lax.gather(v, idx, dn, slice_sizes=(1,),
                      mode=lax.GatherScatterMode.PROMISE_IN_BOUNDS)


def _gate_body(ef_ref, w_ref, out_ref):
    ef = ef_ref[...].reshape(ef_ref.shape[0] // 128, 128)
    z = jnp.dot(ef, w_ref[...], preferred_element_type=jnp.float32)
    out_ref[...] = jax.nn.sigmoid(z).reshape(out_ref.shape)


def _compute_gate(edge_feat, w_gate):
    E, DE = edge_feat.shape
    P = 128
    U = E // 8
    G = 20
    BR = U // G
    assert BR * G == U and BR % 8 == 0
    ef_flat = edge_feat.reshape(U * P)
    wrep = jnp.tile(jnp.kron(jnp.eye(8, dtype=jnp.float32),
                             w_gate.reshape(DE, 1)), (1, 16))
    return pl.pallas_call(
        _gate_body,
        grid=(G,),
        in_specs=[
            pl.BlockSpec((BR * P,), lambda i: (i,)),
            pl.BlockSpec((P, P), lambda i: (0, 0)),
        ],
        out_specs=pl.BlockSpec((BR * P,), lambda i: (i,)),
        out_shape=jax.ShapeDtypeStruct((U * P,), jnp.float32),
    )(ef_flat, wrep)


def _make_sc_agg(N, D, E, e_base):
    NW = _NC * _NS
    EW = E // NW
    C = 80
    CH = EW // C
    RT = 624
    TAIL = N - _NS * RT
    GRP = C // _LANES
    JD = D // _LANES
    ZC = RT // C
    ZT = RT - ZC * C
    assert EW * NW == E and CH * C == EW and CH >= 8
    assert TAIL == 16 and ZT == 64

    mesh = plsc.VectorSubcoreMesh(core_axis_name="c", subcore_axis_name="s")

    @functools.partial(
        pl.kernel,
        out_type=jax.ShapeDtypeStruct((_NC, N, D), jnp.float32),
        mesh=mesh,
        scratch_types=[
            [pltpu.VMEM((16 * C,), jnp.float32) for _ in range(4)],
            [pltpu.VMEM((C,), jnp.int32) for _ in range(4)],
            [pltpu.VMEM((C,), jnp.int32) for _ in range(4)],
            [pltpu.VMEM((C, D), jnp.float32) for _ in range(4)],
            pltpu.VMEM_SHARED((N, D), jnp.float32),
            [pltpu.SemaphoreType.DMA for _ in range(4)],
            [pltpu.SemaphoreType.DMA for _ in range(4)],
            [pltpu.SemaphoreType.DMA for _ in range(4)],
            [pltpu.SemaphoreType.DMA for _ in range(4)],
            [pltpu.SemaphoreType.DMA for _ in range(4)],
        ],
    )
    def sc_agg(src_hbm, dst_hbm, gate_hbm, x_hbm, out_hbm,
               gatev, srcv, dstv, rows, agg_sh,
               sem_gt, sem_src, sem_dst, sem_g, sem_s):
        c = lax.axis_index("c")
        s = lax.axis_index("s")
        wid = s * _NC + c
        base_w = wid * EW
        base_g = e_base + base_w

        zeros = jnp.zeros((_LANES,), jnp.float32)

        def zrow(r, carry):
            for j in range(JD):
                rows[0][r, pl.ds(j * _LANES, _LANES)] = zeros
            return carry

        lax.fori_loop(0, C, zrow, 0)
        for i in range(ZC):
            pltpu.async_copy(rows[0], agg_sh.at[pl.ds(s * RT + i * C, C)], sem_s[i % 4])
        pltpu.async_copy(rows[0].at[pl.ds(0, ZT)],
                         agg_sh.at[pl.ds(s * RT + ZC * C, ZT)], sem_s[3])

        @pl.when(s == 0)
        def _zero_tail():
            pltpu.sync_copy(rows[0].at[pl.ds(0, TAIL)], agg_sh.at[pl.ds(_NS * RT, TAIL)])

        for i in range(ZC):
            pltpu.make_async_copy(rows[0], agg_sh.at[pl.ds(s * RT + i * C, C)],
                                  sem_s[i % 4]).wait()
        pltpu.make_async_copy(rows[0].at[pl.ds(0, ZT)],
                              agg_sh.at[pl.ds(s * RT + ZC * C, ZT)], sem_s[3]).wait()
        plsc.subcore_barrier()

        def start_src(k, p):
            pltpu.async_copy(src_hbm.at[pl.ds(base_g + k * C, C)], srcv[p], sem_src[p])

        def wait_src(k, p):
            pltpu.make_async_copy(src_hbm.at[pl.ds(base_g + k * C, C)],
                                  srcv[p], sem_src[p]).wait()

        def start_dst(k, p):
            pltpu.async_copy(dst_hbm.at[pl.ds(base_g + k * C, C)], dstv[p], sem_dst[p])

        def wait_dst(k, p):
            pltpu.make_async_copy(dst_hbm.at[pl.ds(base_g + k * C, C)],
                                  dstv[p], sem_dst[p]).wait()

        def start_gate(k, p):
            pltpu.async_copy(gate_hbm.at[pl.ds(16 * (base_w + k * C), 16 * C)],
                             gatev[p], sem_gt[p])

        def wait_gate(k, p):
            pltpu.make_async_copy(gate_hbm.at[pl.ds(16 * (base_w + k * C), 16 * C)],
                                  gatev[p], sem_gt[p]).wait()

        def start_gather(p):
            pltpu.async_copy(x_hbm.at[srcv[p]], rows[p], sem_g[p])

        def wait_gather(p):
            pltpu.make_async_copy(x_hbm.at[srcv[p]], rows[p], sem_g[p]).wait()

        def start_scatter(p):
            pltpu.async_copy(rows[p], agg_sh.at[dstv[p]], sem_s[p], add=True)

        def wait_scatter(p):
            pltpu.make_async_copy(rows[p], agg_sh.at[dstv[p]], sem_s[p]).wait()

        def scale(p):
            def grp(q, gcarry):
                ga = gatev[p][pl.ds(q * 256, _LANES)]
                gb = gatev[p][pl.ds(q * 256 + 128, _LANES)]
                for l in range(_LANES):
                    g16 = _lane_bcast(ga if l < 8 else gb, l % 8)
                    e = q * _LANES + l
                    for j in range(JD):
                        sl = pl.ds(j * _LANES, _LANES)
                        rows[p][e, sl] = rows[p][e, sl] * g16
                return gcarry

            lax.fori_loop(0, GRP, grp, 0)

        for j in range(4):
            start_src(j, j)
            start_dst(j, j)
            start_gate(j, j)
        for j in range(4):
            wait_src(j, j)
            start_gather(j)

        def head(k, p):
            wait_gather(p)
            wait_gate(k, p)
            scale(p)
            wait_dst(k, p)
            start_scatter(p)

        head(0, 0)
        start_src(4, 0)
        head(1, 1)
        head(2, 2)

        def full_step(k, p, pn, pn2, with_next2):
            wait_scatter(pn)
            start_dst(k + 1, pn)
            start_gate(k + 1, pn)
            if with_next2:
                start_src(k + 2, pn2)
            wait_src(k + 1, pn)
            start_gather(pn)
            wait_gather(p)
            wait_gate(k, p)
            scale(p)
            wait_dst(k, p)
            start_scatter(p)

        T = (CH - 5) // 4

        def quad(t, carry):
            k0 = 4 * t + 3
            full_step(k0, 3, 0, 1, True)
            full_step(k0 + 1, 0, 1, 2, True)
            full_step(k0 + 2, 1, 2, 3, True)
            full_step(k0 + 3, 2, 3, 0, True)
            return carry

        lax.fori_loop(0, T, quad, 0)
        for k in range(3 + 4 * T, CH):
            p, pn, pn2 = k % 4, (k + 1) % 4, (k + 2) % 4
            if k < CH - 1:
                full_step(k, p, pn, pn2, k + 2 < CH)
            else:
                wait_scatter(pn)
                wait_gather(p)
                wait_gate(k, p)
                scale(p)
                wait_dst(k, p)
                start_scatter(p)
        for k in range(CH - 3, CH):
            wait_scatter(k % 4)
        plsc.subcore_barrier()

        pltpu.sync_copy(agg_sh.at[pl.ds(s * RT, RT)], out_hbm.at[c, pl.ds(s * RT, RT)])

        @pl.when(s == 0)
        def _write_tail():
            pltpu.sync_copy(agg_sh.at[pl.ds(_NS * RT, TAIL)],
                            out_hbm.at[c, pl.ds(_NS * RT, TAIL)])

    return sc_agg


def _out_body(a0_ref, a1_ref, a2_ref, a3_ref, x_ref, wn_ref, ws_ref, b_ref, o_ref):
    agg = (a0_ref[...] + a1_ref[...]) + (a2_ref[...] + a3_ref[...])
    acc = jnp.dot(agg, wn_ref[...], preferred_element_type=jnp.float32)
    acc = acc + jnp.dot(x_ref[...], ws_ref[...], preferred_element_type=jnp.float32)
    acc = acc + b_ref[...]
    o_ref[...] = jnp.maximum(acc, 0.0)


def _compute_out(aggs, x, W_neigh, W_self, b):
    N, D = x.shape
    R = 1000
    G = N // R
    b_row = b.reshape(1, D)
    blk = pl.BlockSpec((R, D), lambda i: (i, 0))
    return pl.pallas_call(
        _out_body,
        grid=(G,),
        in_specs=[
            blk, blk, blk, blk,
            pl.BlockSpec((R, D), lambda i: (i, 0)),
            pl.BlockSpec((D, D), lambda i: (0, 0)),
            pl.BlockSpec((D, D), lambda i: (0, 0)),
            pl.BlockSpec((1, D), lambda i: (0, 0)),
        ],
        out_specs=pl.BlockSpec((R, D), lambda i: (i, 0)),
        out_shape=jax.ShapeDtypeStruct((N, D), jnp.float32),
    )(*aggs, x, W_neigh, W_self, b_row)


@jax.jit
def kernel(g, node_feat, edge_feat, embed_weight, W_self, W_neigh, w_gate, b):
    N, D = embed_weight.shape
    E = edge_feat.shape[0]
    src = g[0]
    dst = g[1]
    E0 = 163840
    parts = []
    for lo, hi in ((0, E0), (E0, E)):
        gate = _compute_gate(edge_feat[lo:hi], w_gate)
        aggs = _make_sc_agg(N, D, hi - lo, lo)(src, dst, gate, embed_weight)
        parts.append(aggs[0])
        parts.append(aggs[1])
    return _compute_out(parts, embed_weight, W_neigh, W_self, b)
